# Initial kernel scaffold; baseline (speedup 1.0000x reference)
import jax
import jax.numpy as jnp
from jax.experimental import pallas as pl


def kernel(hidden_states, primary_attention_output, slot_keys, slot_values, reliability, Wq, Wd, Wu):
    raise NotImplementedError("write your pallas kernel here")


# speedup vs baseline: 6.3745x; 6.3745x over previous
import math

import jax
import jax.numpy as jnp
from jax import lax
from jax.experimental import pallas as pl
from jax.experimental.pallas import tpu as pltpu

_BD = 128
_VB = 256
_N = 256
_TOPK = 8
_T = 256


def _body(hs_ref, prim_ref, skT_ref, sv_ref, rel_ref, wqT_ref, wdT_ref,
          wuT_ref, out_ref):
    scale = 1.0 / math.sqrt(_BD)
    hs = hs_ref[...].astype(jnp.bfloat16)
    q = jnp.dot(hs, wqT_ref[...], preferred_element_type=jnp.float32)
    rel_log = jnp.log(jnp.clip(rel_ref[...], 1e-10, None))
    qk = jnp.dot(q.astype(jnp.bfloat16), skT_ref[...],
                 preferred_element_type=jnp.float32) * scale
    scores = qk + rel_log

    ids = lax.broadcasted_iota(jnp.int32, scores.shape, 1)
    m = scores
    sel = jnp.zeros(scores.shape, dtype=jnp.bool_)
    gate = None
    for k in range(_TOPK):
        cmax = jnp.max(m, axis=-1, keepdims=True)
        if k == 0:
            gate = jax.nn.sigmoid(cmax)
        idx = jnp.min(jnp.where(m == cmax, ids, _N), axis=-1, keepdims=True)
        pick = ids == idx
        sel = jnp.logical_or(sel, pick)
        m = jnp.where(pick, -1e30, m)

    logits = jnp.where(sel, qk, -1e30)
    lmax = jnp.max(logits, axis=-1, keepdims=True)
    p = jnp.exp(logits - lmax) * sel.astype(jnp.float32)
    w = p / jnp.sum(p, axis=-1, keepdims=True)

    aux = jnp.dot(w.astype(jnp.bfloat16), sv_ref[...],
                  preferred_element_type=jnp.float32)
    a = jnp.dot(aux.astype(jnp.bfloat16), wdT_ref[...],
                preferred_element_type=jnp.float32)
    g = jax.nn.gelu(a)
    u = jnp.dot(g.astype(jnp.bfloat16), wuT_ref[...],
                preferred_element_type=jnp.float32)
    out_ref[...] = prim_ref[...] + gate * u


def kernel(hidden_states, primary_attention_output, slot_keys, slot_values,
           reliability, Wq, Wd, Wu):
    B, S, H = hidden_states.shape
    M = B * S
    hs = hidden_states.reshape(M, H)
    prim = primary_attention_output.reshape(M, H)
    skT = slot_keys.T.astype(jnp.bfloat16)
    wqT = Wq.T.astype(jnp.bfloat16)
    wdT = Wd.T.astype(jnp.bfloat16)
    wuT = Wu.T.astype(jnp.bfloat16)
    sv = slot_values.astype(jnp.bfloat16)
    rel = reliability.reshape(1, _N)

    const = lambda i: (0, 0)
    out = pl.pallas_call(
        _body,
        grid=(M // _T,),
        in_specs=[
            pl.BlockSpec((_T, H), lambda i: (i, 0)),
            pl.BlockSpec((_T, H), lambda i: (i, 0)),
            pl.BlockSpec((_BD, _N), const),
            pl.BlockSpec((_N, H), const),
            pl.BlockSpec((1, _N), const),
            pl.BlockSpec((H, _BD), const),
            pl.BlockSpec((H, _VB), const),
            pl.BlockSpec((_VB, H), const),
        ],
        out_specs=pl.BlockSpec((_T, H), lambda i: (i, 0)),
        out_shape=jax.ShapeDtypeStruct((M, H), jnp.float32),
        compiler_params=pltpu.CompilerParams(
            dimension_semantics=("arbitrary",),
        ),
    )(hs, prim, skT, sv, rel, wqT, wdT, wuT)
    return out.reshape(B, S, H)


# speedup vs baseline: 7.7770x; 1.2200x over previous
import math

import jax
import jax.numpy as jnp
from jax import lax
from jax.experimental import pallas as pl
from jax.experimental.pallas import tpu as pltpu

_BD = 128
_VB = 256
_N = 256
_TOPK = 8
_T = 256


def _body(hs_ref, prim_ref, sk_ref, sv_ref, rel_ref, wqT_ref, wdT_ref,
          wuT_ref, out_ref):
    scale = 1.0 / math.sqrt(_BD)
    hs = hs_ref[...].astype(jnp.bfloat16)
    q = jnp.dot(hs, wqT_ref[...], preferred_element_type=jnp.float32)
    rel_log = jnp.log(jnp.clip(rel_ref[...], 1e-10, None))
    qkT = lax.dot_general(sk_ref[...], q.astype(jnp.bfloat16),
                          (((1,), (1,)), ((), ())),
                          preferred_element_type=jnp.float32) * scale
    scoresT = qkT + rel_log

    ids = lax.broadcasted_iota(jnp.int32, scoresT.shape, 0)
    m = scoresT
    sel = jnp.zeros(scoresT.shape, dtype=jnp.bool_)
    gate_row = None
    for k in range(_TOPK):
        cmax = jnp.max(m, axis=0, keepdims=True)
        if k == 0:
            gate_row = jax.nn.sigmoid(cmax)
        idx = jnp.min(jnp.where(m == cmax, ids, _N), axis=0, keepdims=True)
        pick = ids == idx
        sel = jnp.logical_or(sel, pick)
        m = jnp.where(pick, -1e30, m)

    logitsT = jnp.where(sel, qkT, -1e30)
    lmax = jnp.max(logitsT, axis=0, keepdims=True)
    p = jnp.exp(logitsT - lmax) * sel.astype(jnp.float32)
    w = p / jnp.sum(p, axis=0, keepdims=True)

    aux = lax.dot_general(w.astype(jnp.bfloat16), sv_ref[...],
                          (((0,), (0,)), ((), ())),
                          preferred_element_type=jnp.float32)
    a = jnp.dot(aux.astype(jnp.bfloat16), wdT_ref[...],
                preferred_element_type=jnp.float32)
    g = jax.nn.gelu(a)
    u = jnp.dot(g.astype(jnp.bfloat16), wuT_ref[...],
                preferred_element_type=jnp.float32)
    gate = gate_row.reshape(_T, 1)
    out_ref[...] = prim_ref[...] + gate * u


def kernel(hidden_states, primary_attention_output, slot_keys, slot_values,
           reliability, Wq, Wd, Wu):
    B, S, H = hidden_states.shape
    M = B * S
    hs = hidden_states.reshape(M, H)
    prim = primary_attention_output.reshape(M, H)
    sk = slot_keys.astype(jnp.bfloat16)
    wqT = Wq.T.astype(jnp.bfloat16)
    wdT = Wd.T.astype(jnp.bfloat16)
    wuT = Wu.T.astype(jnp.bfloat16)
    sv = slot_values.astype(jnp.bfloat16)
    rel = reliability.reshape(_N, 1)

    const = lambda i: (0, 0)
    out = pl.pallas_call(
        _body,
        grid=(M // _T,),
        in_specs=[
            pl.BlockSpec((_T, H), lambda i: (i, 0)),
            pl.BlockSpec((_T, H), lambda i: (i, 0)),
            pl.BlockSpec((_N, _BD), const),
            pl.BlockSpec((_N, H), const),
            pl.BlockSpec((_N, 1), const),
            pl.BlockSpec((H, _BD), const),
            pl.BlockSpec((H, _VB), const),
            pl.BlockSpec((_VB, H), const),
        ],
        out_specs=pl.BlockSpec((_T, H), lambda i: (i, 0)),
        out_shape=jax.ShapeDtypeStruct((M, H), jnp.float32),
        compiler_params=pltpu.CompilerParams(
            dimension_semantics=("arbitrary",),
        ),
    )(hs, prim, sk, sv, rel, wqT, wdT, wuT)
    return out.reshape(B, S, H)


# speedup vs baseline: 8.0418x; 1.0341x over previous
import math

import jax
import jax.numpy as jnp
from jax import lax
from jax.experimental import pallas as pl
from jax.experimental.pallas import tpu as pltpu

_BD = 128
_VB = 256
_N = 256
_TOPK = 8
_T = 256


def _body(hs_ref, prim_ref, sk_ref, sv_ref, rel_ref, wqT_ref, wdT_ref,
          wuT_ref, out_ref):
    scale = 1.0 / math.sqrt(_BD)
    hs = hs_ref[...].astype(jnp.bfloat16)
    q = jnp.dot(hs, wqT_ref[...], preferred_element_type=jnp.float32)
    rel_log = jnp.log(jnp.clip(rel_ref[...], 1e-10, None))
    qkT = lax.dot_general(sk_ref[...], q.astype(jnp.bfloat16),
                          (((1,), (1,)), ((), ())),
                          preferred_element_type=jnp.float32) * scale
    scoresT = qkT + rel_log

    ids = lax.broadcasted_iota(jnp.int32, scoresT.shape, 0)
    m = scoresT
    gate_row = None
    for k in range(_TOPK):
        cmax = jnp.max(m, axis=0, keepdims=True)
        if k == 0:
            gate_row = jax.nn.sigmoid(cmax)
        idx = jnp.min(jnp.where(m == cmax, ids, _N), axis=0, keepdims=True)
        m = jnp.where(ids == idx, -1e30, m)
    sel = m < -1e29

    logitsT = jnp.where(sel, qkT, -1e30)
    lmax = jnp.max(logitsT, axis=0, keepdims=True)
    p = jnp.exp(logitsT - lmax) * sel.astype(jnp.float32)
    w = p / jnp.sum(p, axis=0, keepdims=True)

    aux = lax.dot_general(w.astype(jnp.bfloat16), sv_ref[...],
                          (((0,), (0,)), ((), ())),
                          preferred_element_type=jnp.float32)
    a = jnp.dot(aux.astype(jnp.bfloat16), wdT_ref[...],
                preferred_element_type=jnp.float32)
    g = jax.nn.gelu(a)
    u = jnp.dot(g.astype(jnp.bfloat16), wuT_ref[...],
                preferred_element_type=jnp.float32)
    gate = gate_row.reshape(_T, 1)
    out_ref[...] = prim_ref[...] + gate * u


def kernel(hidden_states, primary_attention_output, slot_keys, slot_values,
           reliability, Wq, Wd, Wu):
    B, S, H = hidden_states.shape
    M = B * S
    hs = hidden_states.reshape(M, H)
    prim = primary_attention_output.reshape(M, H)
    sk = slot_keys.astype(jnp.bfloat16)
    wqT = Wq.T.astype(jnp.bfloat16)
    wdT = Wd.T.astype(jnp.bfloat16)
    wuT = Wu.T.astype(jnp.bfloat16)
    sv = slot_values.astype(jnp.bfloat16)
    rel = reliability.reshape(_N, 1)

    const = lambda i: (0, 0)
    out = pl.pallas_call(
        _body,
        grid=(M // _T,),
        in_specs=[
            pl.BlockSpec((_T, H), lambda i: (i, 0)),
            pl.BlockSpec((_T, H), lambda i: (i, 0)),
            pl.BlockSpec((_N, _BD), const),
            pl.BlockSpec((_N, H), const),
            pl.BlockSpec((_N, 1), const),
            pl.BlockSpec((H, _BD), const),
            pl.BlockSpec((H, _VB), const),
            pl.BlockSpec((_VB, H), const),
        ],
        out_specs=pl.BlockSpec((_T, H), lambda i: (i, 0)),
        out_shape=jax.ShapeDtypeStruct((M, H), jnp.float32),
        compiler_params=pltpu.CompilerParams(
            dimension_semantics=("arbitrary",),
        ),
    )(hs, prim, sk, sv, rel, wqT, wdT, wuT)
    return out.reshape(B, S, H)


# speedup vs baseline: 9.4009x; 1.1690x over previous
import math

import jax
import jax.numpy as jnp
from jax import lax
from jax.experimental import pallas as pl
from jax.experimental.pallas import tpu as pltpu

_BD = 128
_VB = 256
_N = 256
_TOPK = 8
_T = 256


def _body(hs_ref, prim_ref, sk_ref, sv_ref, rel_ref, wq_ref, wd_ref, wu_ref,
          out_ref, sk_s, sv_s, rel_s, wqT_s, wdT_s, wuT_s):
    i = pl.program_id(0)

    @pl.when(i == 0)
    def _prep():
        sk_s[...] = sk_ref[...].astype(jnp.bfloat16)
        sv_s[...] = sv_ref[...].astype(jnp.bfloat16)
        rel_s[...] = jnp.log(jnp.clip(rel_ref[...], 1e-10, None))
        wqT_s[...] = wq_ref[...].T.astype(jnp.bfloat16)
        wdT_s[...] = wd_ref[...].T.astype(jnp.bfloat16)
        wuT_s[...] = wu_ref[...].T.astype(jnp.bfloat16)

    scale = 1.0 / math.sqrt(_BD)
    hs = hs_ref[...].astype(jnp.bfloat16)
    q = jnp.dot(hs, wqT_s[...], preferred_element_type=jnp.float32)
    qkT = lax.dot_general(sk_s[...], q.astype(jnp.bfloat16),
                          (((1,), (1,)), ((), ())),
                          preferred_element_type=jnp.float32) * scale
    scoresT = qkT + rel_s[...]

    ids = lax.broadcasted_iota(jnp.int32, scoresT.shape, 0)
    m = scoresT
    gate_row = None
    for k in range(_TOPK):
        cmax = jnp.max(m, axis=0, keepdims=True)
        if k == 0:
            gate_row = jax.nn.sigmoid(cmax)
        idx = jnp.min(jnp.where(m == cmax, ids, _N), axis=0, keepdims=True)
        m = jnp.where(ids == idx, -1e30, m)
    sel = m < -1e29

    logitsT = jnp.where(sel, qkT, -1e30)
    lmax = jnp.max(logitsT, axis=0, keepdims=True)
    p = jnp.exp(logitsT - lmax) * sel.astype(jnp.float32)
    w = p / jnp.sum(p, axis=0, keepdims=True)

    aux = lax.dot_general(w.astype(jnp.bfloat16), sv_s[...],
                          (((0,), (0,)), ((), ())),
                          preferred_element_type=jnp.float32)
    a = jnp.dot(aux.astype(jnp.bfloat16), wdT_s[...],
                preferred_element_type=jnp.float32)
    g = jax.nn.gelu(a)
    u = jnp.dot(g.astype(jnp.bfloat16), wuT_s[...],
                preferred_element_type=jnp.float32)
    gate = gate_row.reshape(_T, 1)
    out_ref[...] = prim_ref[...] + gate * u


def kernel(hidden_states, primary_attention_output, slot_keys, slot_values,
           reliability, Wq, Wd, Wu):
    B, S, H = hidden_states.shape
    M = B * S
    hs = hidden_states.reshape(M, H)
    prim = primary_attention_output.reshape(M, H)
    rel = reliability.reshape(_N, 1)

    const = lambda i: (0, 0)
    out = pl.pallas_call(
        _body,
        grid=(M // _T,),
        in_specs=[
            pl.BlockSpec((_T, H), lambda i: (i, 0)),
            pl.BlockSpec((_T, H), lambda i: (i, 0)),
            pl.BlockSpec((_N, _BD), const),
            pl.BlockSpec((_N, H), const),
            pl.BlockSpec((_N, 1), const),
            pl.BlockSpec((_BD, H), const),
            pl.BlockSpec((_VB, H), const),
            pl.BlockSpec((H, _VB), const),
        ],
        out_specs=pl.BlockSpec((_T, H), lambda i: (i, 0)),
        out_shape=jax.ShapeDtypeStruct((M, H), jnp.float32),
        scratch_shapes=[
            pltpu.VMEM((_N, _BD), jnp.bfloat16),
            pltpu.VMEM((_N, H), jnp.bfloat16),
            pltpu.VMEM((_N, 1), jnp.float32),
            pltpu.VMEM((H, _BD), jnp.bfloat16),
            pltpu.VMEM((H, _VB), jnp.bfloat16),
            pltpu.VMEM((_VB, H), jnp.bfloat16),
        ],
        compiler_params=pltpu.CompilerParams(
            dimension_semantics=("arbitrary",),
        ),
    )(hs, prim, slot_keys, slot_values, rel, Wq, Wd, Wu)
    return out.reshape(B, S, H)


# speedup vs baseline: 11.2256x; 1.1941x over previous
import math

import jax
import jax.numpy as jnp
from jax import lax
from jax.experimental import pallas as pl
from jax.experimental.pallas import tpu as pltpu

_BD = 128
_VB = 256
_N = 256
_TOPK = 8
_T = 256


def _body(hs_ref, prim_ref, sk_ref, sv_ref, rel_ref, wq_ref, wd_ref, wu_ref,
          out_ref, sk_s, svd_s, rel_s, wqT_s, wuT_s):
    i = pl.program_id(0)

    @pl.when(i == 0)
    def _prep():
        sk_s[...] = sk_ref[...].astype(jnp.bfloat16)
        rel_s[...] = jnp.log(jnp.clip(rel_ref[...], 1e-10, None))
        wqT_s[...] = wq_ref[...].T.astype(jnp.bfloat16)
        wuT_s[...] = wu_ref[...].T.astype(jnp.bfloat16)
        svd = lax.dot_general(sv_ref[...].astype(jnp.bfloat16),
                              wd_ref[...].astype(jnp.bfloat16),
                              (((1,), (1,)), ((), ())),
                              preferred_element_type=jnp.float32)
        svd_s[...] = svd.astype(jnp.bfloat16)

    scale = 1.0 / math.sqrt(_BD)
    hs = hs_ref[...].astype(jnp.bfloat16)
    q = jnp.dot(hs, wqT_s[...], preferred_element_type=jnp.float32)
    qkT = lax.dot_general(sk_s[...], q.astype(jnp.bfloat16),
                          (((1,), (1,)), ((), ())),
                          preferred_element_type=jnp.float32) * scale
    scoresT = qkT + rel_s[...]

    ids = lax.broadcasted_iota(jnp.int32, scoresT.shape, 0)
    m = scoresT
    gate_row = None
    for k in range(_TOPK):
        cmax = jnp.max(m, axis=0, keepdims=True)
        if k == 0:
            gate_row = jax.nn.sigmoid(cmax)
        idx = jnp.min(jnp.where(m == cmax, ids, _N), axis=0, keepdims=True)
        m = jnp.where(ids == idx, -1e30, m)
    sel = m < -1e29

    logitsT = jnp.where(sel, qkT, -1e30)
    lmax = jnp.max(logitsT, axis=0, keepdims=True)
    p = jnp.exp(logitsT - lmax) * sel.astype(jnp.float32)
    w = p / jnp.sum(p, axis=0, keepdims=True)

    a = lax.dot_general(w.astype(jnp.bfloat16), svd_s[...],
                        (((0,), (0,)), ((), ())),
                        preferred_element_type=jnp.float32)
    g = jax.nn.gelu(a)
    u = jnp.dot(g.astype(jnp.bfloat16), wuT_s[...],
                preferred_element_type=jnp.float32)
    gate = gate_row.reshape(_T, 1)
    out_ref[...] = prim_ref[...] + gate * u


def kernel(hidden_states, primary_attention_output, slot_keys, slot_values,
           reliability, Wq, Wd, Wu):
    B, S, H = hidden_states.shape
    M = B * S
    hs = hidden_states.reshape(M, H)
    prim = primary_attention_output.reshape(M, H)
    rel = reliability.reshape(_N, 1)

    const = lambda i: (0, 0)
    out = pl.pallas_call(
        _body,
        grid=(M // _T,),
        in_specs=[
            pl.BlockSpec((_T, H), lambda i: (i, 0)),
            pl.BlockSpec((_T, H), lambda i: (i, 0)),
            pl.BlockSpec((_N, _BD), const),
            pl.BlockSpec((_N, H), const),
            pl.BlockSpec((_N, 1), const),
            pl.BlockSpec((_BD, H), const),
            pl.BlockSpec((_VB, H), const),
            pl.BlockSpec((H, _VB), const),
        ],
        out_specs=pl.BlockSpec((_T, H), lambda i: (i, 0)),
        out_shape=jax.ShapeDtypeStruct((M, H), jnp.float32),
        scratch_shapes=[
            pltpu.VMEM((_N, _BD), jnp.bfloat16),
            pltpu.VMEM((_N, _VB), jnp.bfloat16),
            pltpu.VMEM((_N, 1), jnp.float32),
            pltpu.VMEM((H, _BD), jnp.bfloat16),
            pltpu.VMEM((_VB, H), jnp.bfloat16),
        ],
        compiler_params=pltpu.CompilerParams(
            dimension_semantics=("arbitrary",),
        ),
    )(hs, prim, slot_keys, slot_values, rel, Wq, Wd, Wu)
    return out.reshape(B, S, H)
